# baseline (speedup 1.0000x reference)
import functools

import jax
import jax.numpy as jnp
from jax import lax
from jax.experimental import pallas as pl
from jax.experimental.pallas import tpu as pltpu
from jax.experimental.pallas import tpu_sc as plsc

N = 10000
E = 160000
D = 256
H = 512
NP = 10240
EPAD = 163840
NS = 16
NC = 2
EPT = EPAD // NS
CH = 128
CHA = 64
NB = EPT // CH
NBA = EPT // CHA
RPT = NP // NS

_mesh = plsc.VectorSubcoreMesh(core_axis_name="c", subcore_axis_name="s")
_sc_params = pltpu.CompilerParams(needs_layout_passes=False)


def _newton_rsqrt(d):
    bits = lax.bitcast_convert_type(d, jnp.int32)
    y = lax.bitcast_convert_type(
        jnp.int32(0x5F3759DF) - lax.shift_right_logical(bits, 1), jnp.float32)
    for _ in range(3):
        y = y * (1.5 - 0.5 * d * y * y)
    return y


@functools.partial(
    pl.kernel,
    mesh=_mesh,
    compiler_params=_sc_params,
    out_type=(jax.ShapeDtypeStruct((NP,), jnp.float32),
              jax.ShapeDtypeStruct((NS, EPT), jnp.float32)),
    scratch_types=[
        pltpu.VMEM((NB, CH), jnp.int32),
        pltpu.VMEM((EPT,), jnp.int32),
        pltpu.VMEM((EPT,), jnp.float32),
        pltpu.VMEM((RPT,), jnp.float32),
        pltpu.VMEM((NP,), jnp.float32),
        pltpu.VMEM_SHARED((NP,), jnp.float32),
    ],
)
def _deg_kernel(dst_hbm, src_hbm, w_hbm, dinv_hbm, coef_hbm,
                dstb, srcb, wb, stage, dv, acc):
    c = lax.axis_index("c")
    s = lax.axis_index("s")

    def zero_body(i, _):
        stage[pl.ds(i * 16, 16)] = jnp.zeros((16,), jnp.float32)
        return 0
    lax.fori_loop(0, RPT // 16, zero_body, 0)
    pltpu.sync_copy(stage.at[pl.ds(0, RPT)], acc.at[pl.ds(s * RPT, RPT)])
    plsc.subcore_barrier()

    pltpu.sync_copy(dst_hbm.at[s], dstb)
    pltpu.sync_copy(w_hbm.at[s], wb)

    def scat_body(b, _):
        pltpu.sync_copy(wb.at[pl.ds(b * CH, CH)], acc.at[dstb.at[b]], add=True)
        return 0
    lax.fori_loop(0, NB, scat_body, 0)
    plsc.subcore_barrier()

    pltpu.sync_copy(acc.at[pl.ds(s * RPT, RPT)], stage.at[pl.ds(0, RPT)])

    def rsq_body(i, _):
        d = stage[pl.ds(i * 16, 16)] + 1.0
        stage[pl.ds(i * 16, 16)] = _newton_rsqrt(d)
        return 0
    lax.fori_loop(0, RPT // 16, rsq_body, 0)
    pltpu.sync_copy(stage.at[pl.ds(0, RPT)], acc.at[pl.ds(s * RPT, RPT)])

    @pl.when(c == 0)
    def _():
        pltpu.sync_copy(stage.at[pl.ds(0, RPT)], dinv_hbm.at[pl.ds(s * RPT, RPT)])
    plsc.subcore_barrier()

    @pl.when(c == 0)
    def _():
        pltpu.sync_copy(acc.at[pl.ds(0, NP)], dv.at[pl.ds(0, NP)])
        pltpu.sync_copy(src_hbm.at[s], srcb)

        def coef_body(j, _):
            s16 = srcb[pl.ds(j * 16, 16)]
            wb[pl.ds(j * 16, 16)] = (wb[pl.ds(j * 16, 16)]
                                     * plsc.load_gather(dv, [s16]))
            return 0
        lax.fori_loop(0, EPT // 16, coef_body, 0)
        pltpu.sync_copy(wb.at[pl.ds(0, EPT)], coef_hbm.at[s])


PH = 8


@functools.partial(
    pl.kernel,
    mesh=_mesh,
    compiler_params=_sc_params,
    out_type=jax.ShapeDtypeStruct((NC, NP, 128), jnp.float32),
    scratch_types=[
        pltpu.VMEM((EPT // PH,), jnp.int32),
        pltpu.VMEM((NBA // PH, CHA), jnp.int32),
        pltpu.VMEM((EPT // PH,), jnp.float32),
        [pltpu.VMEM((CHA, 128), jnp.float32)] * 4,
        pltpu.VMEM((8, 128), jnp.float32),
        pltpu.VMEM_SHARED((NP, 128), jnp.float32),
        [pltpu.SemaphoreType.DMA] * 4,
        [pltpu.SemaphoreType.DMA] * 4,
    ],
)
def _agg_kernel(tab_hbm, src_hbm, dst_hbm, coef_hbm, out_hbm,
                srcb, dstb, cb, gbufs, zb, acc, sgs, sss):
    c = lax.axis_index("c")
    s = lax.axis_index("s")
    nbh = NBA // PH
    eph = EPT // PH

    def zrow(i, _):
        for k in range(8):
            zb[i, pl.ds(k * 16, 16)] = jnp.zeros((16,), jnp.float32)
        return 0
    lax.fori_loop(0, 8, zrow, 0)

    def zcopy(i, _):
        pltpu.sync_copy(zb.at[pl.ds(0, 8)],
                        acc.at[pl.ds(s * RPT + i * 8, 8)])
        return 0
    lax.fori_loop(0, RPT // 8, zcopy, 0)
    plsc.subcore_barrier()

    def gather_to(b, buf, sem):
        return tab_hbm.at[srcb.at[pl.ds(b * CHA, CHA)]], buf, sem

    for ph in range(PH):
        pltpu.sync_copy(src_hbm.at[c, s, pl.ds(ph * eph, eph)], srcb)
        pltpu.sync_copy(dst_hbm.at[s, ph], dstb)
        pltpu.sync_copy(coef_hbm.at[s, pl.ds(ph * eph, eph)], cb)

        pltpu.async_copy(*gather_to(0, gbufs[0], sgs[0]))
        pltpu.async_copy(*gather_to(1, gbufs[1], sgs[1]))

        def process(b, j):
            jn = (j + 2) % 4

            @pl.when(b >= 2)
            def _():
                pltpu.make_async_copy(gbufs[jn], acc.at[dstb.at[b - 2]],
                                      sss[jn]).wait()

            @pl.when(b + 2 < nbh)
            def _():
                pltpu.async_copy(*gather_to(b + 2, gbufs[jn], sgs[jn]))

            gc = gbufs[j]
            pltpu.make_async_copy(*gather_to(b, gc, sgs[j])).wait()

            def group_body(gi, _):
                w16 = cb[pl.ds(b * CHA + gi * 16, 16)]
                for e in range(16):
                    row = gi * 16 + e
                    wbc = jnp.full((16,), w16[e], jnp.float32)
                    for k in range(8):
                        gc[row, pl.ds(k * 16, 16)] = (
                            gc[row, pl.ds(k * 16, 16)] * wbc)
                return 0
            lax.fori_loop(0, CHA // 16, group_body, 0)

            pltpu.async_copy(gc, acc.at[dstb.at[b]], sss[j], add=True)

        def quad_body(b4, _):
            for j in range(4):
                process(4 * b4 + j, j)
            return 0
        lax.fori_loop(0, nbh // 4, quad_body, 0)
        pltpu.make_async_copy(gbufs[2], acc.at[dstb.at[nbh - 2]],
                              sss[2]).wait()
        pltpu.make_async_copy(gbufs[3], acc.at[dstb.at[nbh - 1]],
                              sss[3]).wait()
    plsc.subcore_barrier()

    pltpu.sync_copy(acc.at[pl.ds(s * RPT, RPT)],
                    out_hbm.at[c, pl.ds(s * RPT, RPT)])


def _mid_body(s1_ref, x_ref, dv_ref, w1_ref, b1_ref, w2_ref, out_ref):
    dv = dv_ref[...]
    s1 = jnp.concatenate([s1_ref[0], s1_ref[1]], axis=1)
    x = jnp.concatenate([x_ref[0], x_ref[1]], axis=1)
    a1 = dv[:, None] * s1 + (dv * dv)[:, None] * x
    h1 = jnp.maximum(
        jnp.dot(a1, w1_ref[...], preferred_element_type=jnp.float32)
        + b1_ref[...][None, :], 0.0)
    g = jnp.dot(h1, w2_ref[...], preferred_element_type=jnp.float32)
    out_ref[0] = g[:, :128]
    out_ref[1] = g[:, 128:]


def _mid_call(s1, xh3, dinv, W1, b1, W2):
    blk = 1024
    nblk = NP // blk
    return pl.pallas_call(
        _mid_body,
        grid=(nblk,),
        in_specs=[
            pl.BlockSpec((NC, blk, 128), lambda i: (0, i, 0)),
            pl.BlockSpec((NC, blk, 128), lambda i: (0, i, 0)),
            pl.BlockSpec((blk,), lambda i: (i,)),
            pl.BlockSpec((D, H), lambda i: (0, 0)),
            pl.BlockSpec((H,), lambda i: (0,)),
            pl.BlockSpec((H, D), lambda i: (0, 0)),
        ],
        out_specs=pl.BlockSpec((NC, blk, 128), lambda i: (0, i, 0)),
        out_shape=jax.ShapeDtypeStruct((NC, NP, 128), jnp.float32),
    )(s1, xh3, dinv, W1, b1, W2)


def _fin_body(s2_ref, g_ref, dv_ref, b2_ref, out_ref):
    dv = dv_ref[...]
    s2 = jnp.concatenate([s2_ref[0], s2_ref[1]], axis=1)
    g = jnp.concatenate([g_ref[0], g_ref[1]], axis=1)
    out_ref[...] = jnp.maximum(
        dv[:, None] * s2 + (dv * dv)[:, None] * g + b2_ref[...][None, :], 0.0)


def _fin_call(s2, g2, dinv, b2):
    blk = 1024
    nblk = NP // blk
    return pl.pallas_call(
        _fin_body,
        grid=(nblk,),
        in_specs=[
            pl.BlockSpec((NC, blk, 128), lambda i: (0, i, 0)),
            pl.BlockSpec((NC, blk, 128), lambda i: (0, i, 0)),
            pl.BlockSpec((blk,), lambda i: (i,)),
            pl.BlockSpec((D,), lambda i: (0,)),
        ],
        out_specs=pl.BlockSpec((blk, D), lambda i: (i, 0)),
        out_shape=jax.ShapeDtypeStruct((NP, D), jnp.float32),
    )(s2, g2, dinv, b2)


def kernel(x, edge_index, edge_weight, W1, b1, W2, b2):
    src = edge_index[0].astype(jnp.int32)
    dst = edge_index[1].astype(jnp.int32)
    w = edge_weight.astype(jnp.float32)

    pad = EPAD - E
    srcp = jnp.concatenate([src, jnp.zeros((pad,), jnp.int32)])
    dstp = jnp.concatenate([dst, jnp.zeros((pad,), jnp.int32)])
    wp = jnp.concatenate([w, jnp.zeros((pad,), jnp.float32)])

    srcR = srcp.reshape(NS, EPT)
    src2 = jnp.stack([srcp, srcp + NP]).reshape(NC, NS, EPT)
    dst3 = dstp.reshape(NS, NB, CH)
    dstA = dstp.reshape(NS, PH, NBA // PH, CHA)
    w2 = wp.reshape(NS, EPT)

    xp = jnp.concatenate([x, jnp.zeros((NP - N, D), jnp.float32)])
    xh = jnp.concatenate([xp[:, :128], xp[:, 128:]])


    dinv, coef = _deg_kernel(dst3, srcR, w2)

    s1 = _agg_kernel(xh, src2, dstA, coef)
    g2 = _mid_call(s1, xh.reshape(NC, NP, 128), dinv, W1, b1, W2)
    s2 = _agg_kernel(g2.reshape(NC * NP, 128), src2, dstA, coef)
    out = _fin_call(s2, g2, dinv, b2)
    return out[:N]


# speedup vs baseline: 1.0564x; 1.0564x over previous
import functools

import jax
import jax.numpy as jnp
from jax import lax
from jax.experimental import pallas as pl
from jax.experimental.pallas import tpu as pltpu
from jax.experimental.pallas import tpu_sc as plsc

N = 10000
E = 160000
D = 256
H = 512
NP = 10240
EPAD = 163840
NS = 16
NC = 2
EPT = EPAD // NS
CH = 128
CHA = 32
NB = EPT // CH
NBA = EPT // CHA
RPT = NP // NS

_mesh = plsc.VectorSubcoreMesh(core_axis_name="c", subcore_axis_name="s")
_sc_params = pltpu.CompilerParams(needs_layout_passes=False)


def _newton_rsqrt(d):
    bits = lax.bitcast_convert_type(d, jnp.int32)
    y = lax.bitcast_convert_type(
        jnp.int32(0x5F3759DF) - lax.shift_right_logical(bits, 1), jnp.float32)
    for _ in range(3):
        y = y * (1.5 - 0.5 * d * y * y)
    return y


@functools.partial(
    pl.kernel,
    mesh=_mesh,
    compiler_params=_sc_params,
    out_type=(jax.ShapeDtypeStruct((NP,), jnp.float32),
              jax.ShapeDtypeStruct((NS, EPT), jnp.float32)),
    scratch_types=[
        pltpu.VMEM((NB, CH), jnp.int32),
        pltpu.VMEM((EPT,), jnp.int32),
        pltpu.VMEM((EPT,), jnp.float32),
        pltpu.VMEM((RPT,), jnp.float32),
        pltpu.VMEM((NP,), jnp.float32),
        pltpu.VMEM_SHARED((NP,), jnp.float32),
    ],
)
def _deg_kernel(dst_hbm, src_hbm, w_hbm, dinv_hbm, coef_hbm,
                dstb, srcb, wb, stage, dv, acc):
    c = lax.axis_index("c")
    s = lax.axis_index("s")

    def zero_body(i, _):
        stage[pl.ds(i * 16, 16)] = jnp.zeros((16,), jnp.float32)
        return 0
    lax.fori_loop(0, RPT // 16, zero_body, 0)
    pltpu.sync_copy(stage.at[pl.ds(0, RPT)], acc.at[pl.ds(s * RPT, RPT)])
    plsc.subcore_barrier()

    pltpu.sync_copy(dst_hbm.at[s], dstb)
    pltpu.sync_copy(w_hbm.at[s], wb)

    def scat_body(b, _):
        pltpu.sync_copy(wb.at[pl.ds(b * CH, CH)], acc.at[dstb.at[b]], add=True)
        return 0
    lax.fori_loop(0, NB, scat_body, 0)
    plsc.subcore_barrier()

    pltpu.sync_copy(acc.at[pl.ds(s * RPT, RPT)], stage.at[pl.ds(0, RPT)])

    def rsq_body(i, _):
        d = stage[pl.ds(i * 16, 16)] + 1.0
        stage[pl.ds(i * 16, 16)] = _newton_rsqrt(d)
        return 0
    lax.fori_loop(0, RPT // 16, rsq_body, 0)
    pltpu.sync_copy(stage.at[pl.ds(0, RPT)], acc.at[pl.ds(s * RPT, RPT)])

    @pl.when(c == 0)
    def _():
        pltpu.sync_copy(stage.at[pl.ds(0, RPT)], dinv_hbm.at[pl.ds(s * RPT, RPT)])
    plsc.subcore_barrier()

    @pl.when(c == 0)
    def _():
        pltpu.sync_copy(acc.at[pl.ds(0, NP)], dv.at[pl.ds(0, NP)])
        pltpu.sync_copy(src_hbm.at[s], srcb)

        def coef_body(j, _):
            s16 = srcb[pl.ds(j * 16, 16)]
            wb[pl.ds(j * 16, 16)] = (wb[pl.ds(j * 16, 16)]
                                     * plsc.load_gather(dv, [s16]))
            return 0
        lax.fori_loop(0, EPT // 16, coef_body, 0)
        pltpu.sync_copy(wb.at[pl.ds(0, EPT)], coef_hbm.at[s])


PH = 2


@functools.partial(
    pl.kernel,
    mesh=_mesh,
    compiler_params=_sc_params,
    out_type=jax.ShapeDtypeStruct((NC, NP, 128), jnp.float32),
    scratch_types=[
        pltpu.VMEM((EPT // PH,), jnp.int32),
        pltpu.VMEM((NBA // PH, CHA), jnp.int32),
        pltpu.VMEM((EPT // PH,), jnp.float32),
        [pltpu.VMEM((CHA, 128), jnp.float32)] * 4,
        pltpu.VMEM((8, 128), jnp.float32),
        pltpu.VMEM_SHARED((NP, 128), jnp.float32),
        [pltpu.SemaphoreType.DMA] * 4,
        [pltpu.SemaphoreType.DMA] * 4,
    ],
)
def _agg_kernel(tab_hbm, src_hbm, dst_hbm, coef_hbm, out_hbm,
                srcb, dstb, cb, gbufs, zb, acc, sgs, sss):
    c = lax.axis_index("c")
    s = lax.axis_index("s")
    nbh = NBA // PH
    eph = EPT // PH

    def zrow(i, _):
        for k in range(8):
            zb[i, pl.ds(k * 16, 16)] = jnp.zeros((16,), jnp.float32)
        return 0
    lax.fori_loop(0, 8, zrow, 0)

    def zcopy(i, _):
        pltpu.sync_copy(zb.at[pl.ds(0, 8)],
                        acc.at[pl.ds(s * RPT + i * 8, 8)])
        return 0
    lax.fori_loop(0, RPT // 8, zcopy, 0)
    plsc.subcore_barrier()

    def gather_to(b, buf, sem):
        return tab_hbm.at[srcb.at[pl.ds(b * CHA, CHA)]], buf, sem

    for ph in range(PH):
        pltpu.sync_copy(src_hbm.at[c, s, pl.ds(ph * eph, eph)], srcb)
        pltpu.sync_copy(dst_hbm.at[s, ph], dstb)
        pltpu.sync_copy(coef_hbm.at[s, pl.ds(ph * eph, eph)], cb)

        pltpu.async_copy(*gather_to(0, gbufs[0], sgs[0]))
        pltpu.async_copy(*gather_to(1, gbufs[1], sgs[1]))

        def process(b, j):
            jn = (j + 2) % 4

            @pl.when(b >= 2)
            def _():
                pltpu.make_async_copy(gbufs[jn], acc.at[dstb.at[b - 2]],
                                      sss[jn]).wait()

            @pl.when(b + 2 < nbh)
            def _():
                pltpu.async_copy(*gather_to(b + 2, gbufs[jn], sgs[jn]))

            gc = gbufs[j]
            pltpu.make_async_copy(*gather_to(b, gc, sgs[j])).wait()

            def group_body(gi, _):
                w16 = cb[pl.ds(b * CHA + gi * 16, 16)]
                for e in range(16):
                    row = gi * 16 + e
                    wbc = jnp.full((16,), w16[e], jnp.float32)
                    for k in range(8):
                        gc[row, pl.ds(k * 16, 16)] = (
                            gc[row, pl.ds(k * 16, 16)] * wbc)
                return 0
            lax.fori_loop(0, CHA // 16, group_body, 0)

            pltpu.async_copy(gc, acc.at[dstb.at[b]], sss[j], add=True)

        def quad_body(b4, _):
            for j in range(4):
                process(4 * b4 + j, j)
            return 0
        lax.fori_loop(0, nbh // 4, quad_body, 0)
        pltpu.make_async_copy(gbufs[2], acc.at[dstb.at[nbh - 2]],
                              sss[2]).wait()
        pltpu.make_async_copy(gbufs[3], acc.at[dstb.at[nbh - 1]],
                              sss[3]).wait()
    plsc.subcore_barrier()

    pltpu.sync_copy(acc.at[pl.ds(s * RPT, RPT)],
                    out_hbm.at[c, pl.ds(s * RPT, RPT)])


def _mid_body(s1_ref, x_ref, dv_ref, w1_ref, b1_ref, w2_ref, out_ref):
    dv = dv_ref[...]
    s1 = jnp.concatenate([s1_ref[0], s1_ref[1]], axis=1)
    x = jnp.concatenate([x_ref[0], x_ref[1]], axis=1)
    a1 = dv[:, None] * s1 + (dv * dv)[:, None] * x
    h1 = jnp.maximum(
        jnp.dot(a1, w1_ref[...], preferred_element_type=jnp.float32)
        + b1_ref[...][None, :], 0.0)
    g = jnp.dot(h1, w2_ref[...], preferred_element_type=jnp.float32)
    out_ref[0] = g[:, :128]
    out_ref[1] = g[:, 128:]


def _mid_call(s1, xh3, dinv, W1, b1, W2):
    blk = 1024
    nblk = NP // blk
    return pl.pallas_call(
        _mid_body,
        grid=(nblk,),
        in_specs=[
            pl.BlockSpec((NC, blk, 128), lambda i: (0, i, 0)),
            pl.BlockSpec((NC, blk, 128), lambda i: (0, i, 0)),
            pl.BlockSpec((blk,), lambda i: (i,)),
            pl.BlockSpec((D, H), lambda i: (0, 0)),
            pl.BlockSpec((H,), lambda i: (0,)),
            pl.BlockSpec((H, D), lambda i: (0, 0)),
        ],
        out_specs=pl.BlockSpec((NC, blk, 128), lambda i: (0, i, 0)),
        out_shape=jax.ShapeDtypeStruct((NC, NP, 128), jnp.float32),
    )(s1, xh3, dinv, W1, b1, W2)


def _fin_body(s2_ref, g_ref, dv_ref, b2_ref, out_ref):
    dv = dv_ref[...]
    s2 = jnp.concatenate([s2_ref[0], s2_ref[1]], axis=1)
    g = jnp.concatenate([g_ref[0], g_ref[1]], axis=1)
    out_ref[...] = jnp.maximum(
        dv[:, None] * s2 + (dv * dv)[:, None] * g + b2_ref[...][None, :], 0.0)


def _fin_call(s2, g2, dinv, b2):
    blk = 1024
    nblk = NP // blk
    return pl.pallas_call(
        _fin_body,
        grid=(nblk,),
        in_specs=[
            pl.BlockSpec((NC, blk, 128), lambda i: (0, i, 0)),
            pl.BlockSpec((NC, blk, 128), lambda i: (0, i, 0)),
            pl.BlockSpec((blk,), lambda i: (i,)),
            pl.BlockSpec((D,), lambda i: (0,)),
        ],
        out_specs=pl.BlockSpec((blk, D), lambda i: (i, 0)),
        out_shape=jax.ShapeDtypeStruct((NP, D), jnp.float32),
    )(s2, g2, dinv, b2)


def kernel(x, edge_index, edge_weight, W1, b1, W2, b2):
    src = edge_index[0].astype(jnp.int32)
    dst = edge_index[1].astype(jnp.int32)
    w = edge_weight.astype(jnp.float32)

    pad = EPAD - E
    srcp = jnp.concatenate([src, jnp.zeros((pad,), jnp.int32)])
    dstp = jnp.concatenate([dst, jnp.zeros((pad,), jnp.int32)])
    wp = jnp.concatenate([w, jnp.zeros((pad,), jnp.float32)])

    srcR = srcp.reshape(NS, EPT)
    src2 = jnp.stack([srcp, srcp + NP]).reshape(NC, NS, EPT)
    dst3 = dstp.reshape(NS, NB, CH)
    dstA = dstp.reshape(NS, PH, NBA // PH, CHA)
    w2 = wp.reshape(NS, EPT)

    xp = jnp.concatenate([x, jnp.zeros((NP - N, D), jnp.float32)])
    xh = jnp.concatenate([xp[:, :128], xp[:, 128:]])


    dinv, coef = _deg_kernel(dst3, srcR, w2)

    s1 = _agg_kernel(xh, src2, dstA, coef)
    g2 = _mid_call(s1, xh.reshape(NC, NP, 128), dinv, W1, b1, W2)
    s2 = _agg_kernel(g2.reshape(NC * NP, 128), src2, dstA, coef)
    out = _fin_call(s2, g2, dinv, b2)
    return out[:N]
